# baseline (speedup 1.0000x reference)
import functools

import jax
import jax.numpy as jnp
from jax import lax
from jax.experimental import pallas as pl
from jax.experimental.pallas import tpu as pltpu
from jax.experimental.pallas import tpu_sc as plsc

EMBED_DIM = 32
BATCH = 4096
HIST_LEN = 200
B_TOTAL = BATCH * HIST_LEN

_info = plsc.get_sparse_core_info()
_NC, _NS = _info.num_cores, _info.num_subcores
_NW = _NC * _NS
_B_PER_W = B_TOTAL // _NW
_CHUNK = 1600
_N_CHUNKS = _B_PER_W // _CHUNK

_mesh = plsc.VectorSubcoreMesh(core_axis_name="c", subcore_axis_name="s")


@functools.partial(
    pl.kernel,
    mesh=_mesh,
    out_type=jax.ShapeDtypeStruct((B_TOTAL, EMBED_DIM), jnp.float32),
    scratch_types=[
        pltpu.VMEM((_CHUNK,), jnp.int32),
        pltpu.VMEM((_CHUNK,), jnp.int32),
        pltpu.VMEM((_CHUNK, EMBED_DIM), jnp.float32),
        pltpu.VMEM((_CHUNK, EMBED_DIM), jnp.float32),
        pltpu.SemaphoreType.DMA,
        pltpu.SemaphoreType.DMA,
        pltpu.SemaphoreType.DMA,
        pltpu.SemaphoreType.DMA,
    ],
    compiler_params=pltpu.CompilerParams(use_tc_tiling_on_sc=False),
)
def _sc_gather(idx_hbm, table_hbm, out_hbm, idx0, idx1, rows0, rows1,
               gsem0, gsem1, osem0, osem1):
    wid = lax.axis_index("s") * _NC + lax.axis_index("c")
    base = wid * _B_PER_W

    idx_v = (idx0, idx1)
    rows_v = (rows0, rows1)
    gsem = (gsem0, gsem1)
    osem = (osem0, osem1)

    gat = [None, None]
    outw = [None, None]
    for g in range(_N_CHUNKS):
        b = g % 2
        off = base + g * _CHUNK
        if outw[b] is not None:
            outw[b].wait()
        pltpu.sync_copy(idx_hbm.at[pl.ds(off, _CHUNK)], idx_v[b])
        gat[b] = pltpu.async_copy(table_hbm.at[idx_v[b]], rows_v[b], gsem[b])
        if g >= 1:
            p = 1 - b
            gat[p].wait()
            prev_off = base + (g - 1) * _CHUNK
            outw[p] = pltpu.async_copy(
                rows_v[p], out_hbm.at[pl.ds(prev_off, _CHUNK)], osem[p])
    last = (_N_CHUNKS - 1) % 2
    gat[last].wait()
    pltpu.sync_copy(rows_v[last], out_hbm.at[pl.ds(base + (_N_CHUNKS - 1) * _CHUNK, _CHUNK)])
    outw[1 - last].wait()


def kernel(x, table):
    idx = x.reshape(-1).astype(jnp.int32)
    out = _sc_gather(idx, table)
    return out.reshape(BATCH, HIST_LEN, EMBED_DIM)
